# Initial kernel scaffold; baseline (speedup 1.0000x reference)
import jax
import jax.numpy as jnp
from jax.experimental import pallas as pl


def kernel(x, edge_index, W1, b1, W2, b2):
    raise NotImplementedError("write your pallas kernel here")


# speedup vs baseline: 23.7245x; 23.7245x over previous
import functools

import jax
import jax.numpy as jnp
from jax import lax
from jax.experimental import pallas as pl
from jax.experimental.pallas import tpu as pltpu
from jax.experimental.pallas import tpu_sc as plsc

N = 10000
E = 320000
D_IN, D_HID, D_OUT = 128, 64, 6
NC, NS = 2, 16
NW = NC * NS
N_PAD = 10240
ROWS_T = N_PAD // NS
E_W = E // NW
B = 100
NCH = E_W // B
BN = 1024
NB = N_PAD // BN


@functools.lru_cache(maxsize=None)
def _make_spass(d):
    mesh = plsc.VectorSubcoreMesh(
        core_axis_name="c", subcore_axis_name="s", num_cores=NC, num_subcores=NS
    )

    @functools.partial(
        pl.kernel,
        out_type=jax.ShapeDtypeStruct((NC * N_PAD, d), jnp.float32),
        mesh=mesh,
        compiler_params=pltpu.CompilerParams(use_tc_tiling_on_sc=False),
        scratch_types=[
            pltpu.VMEM((NCH, B), jnp.int32),
            pltpu.VMEM((NCH, B), jnp.int32),
            pltpu.VMEM((B, d), jnp.float32),
            pltpu.VMEM((B, d), jnp.float32),
            pltpu.VMEM_SHARED((N_PAD, d), jnp.float32),
            pltpu.SemaphoreType.DMA,
            pltpu.SemaphoreType.DMA,
        ],
    )
    def spass(src3, dst3, rows, zeros, out, src_v, dst_v, rows_a, rows_b,
              acc, sem_a, sem_b):
        c = lax.axis_index("c")
        s = lax.axis_index("s")
        wid = c * NS + s
        row0 = s * ROWS_T

        pltpu.sync_copy(zeros.at[pl.ds(row0, ROWS_T)], acc.at[pl.ds(row0, ROWS_T)])
        pltpu.sync_copy(src3.at[wid], src_v)
        pltpu.sync_copy(dst3.at[wid], dst_v)
        plsc.subcore_barrier()

        def gather(k, buf, sem):
            return pltpu.make_async_copy(rows.at[src_v.at[k]], buf, sem)

        gather(0, rows_a, sem_a).start()

        def step(i, carry):
            ka = 2 * i
            kb = 2 * i + 1
            gather(ka, rows_a, sem_a).wait()
            gather(kb, rows_b, sem_b).start()
            pltpu.sync_copy(rows_a, acc.at[dst_v.at[ka]], add=True)
            gather(kb, rows_b, sem_b).wait()

            @pl.when(i + 1 < NCH // 2)
            def _():
                gather(ka + 2, rows_a, sem_a).start()

            pltpu.sync_copy(rows_b, acc.at[dst_v.at[kb]], add=True)
            return carry

        lax.fori_loop(0, NCH // 2, step, 0)
        plsc.subcore_barrier()
        pltpu.sync_copy(
            acc.at[pl.ds(row0, ROWS_T)],
            out.at[pl.ds(c * N_PAD + row0, ROWS_T)],
        )

    return spass


def _row_spec(d, shift=0):
    return pl.BlockSpec((BN, d), lambda i, _s=shift: (i + _s, 0))


def _full_spec(shape):
    nd = len(shape)
    return pl.BlockSpec(shape, lambda i, _nd=nd: (0,) * nd)


def _tc_call(body, in_specs, out_shapes, out_specs):
    return pl.pallas_call(
        body,
        grid=(NB,),
        in_specs=in_specs,
        out_shape=out_shapes,
        out_specs=out_specs,
    )


def _tc_a(degp, x, w1):
    def body(p0_ref, p1_ref, x_ref, w1_ref, dinv_ref, u_ref):
        deg = 1.0 + p0_ref[:, 0:1] + p1_ref[:, 0:1]
        dinv = lax.rsqrt(deg)
        y1 = jnp.dot(x_ref[...], w1_ref[...], preferred_element_type=jnp.float32)
        dinv_ref[...] = dinv
        u_ref[...] = dinv * y1

    return _tc_call(
        body,
        [_row_spec(16), _row_spec(16, NB), _row_spec(D_IN), _full_spec((D_IN, D_HID))],
        (jax.ShapeDtypeStruct((N_PAD, 1), jnp.float32),
         jax.ShapeDtypeStruct((N_PAD, D_HID), jnp.float32)),
        (_row_spec(1), _row_spec(D_HID)),
    )(degp, degp, x, w1)


def _tc_mid(s, u, dinv, d):
    def body(s0_ref, s1_ref, u_ref, dinv_ref, o_ref):
        dinv = dinv_ref[...]
        o_ref[...] = dinv * dinv * (s0_ref[...] + s1_ref[...] + u_ref[...])

    return _tc_call(
        body,
        [_row_spec(d), _row_spec(d, NB), _row_spec(d), _row_spec(1)],
        jax.ShapeDtypeStruct((N_PAD, d), jnp.float32),
        _row_spec(d),
    )(s, s, u, dinv)


def _tc_c(s, u2, dinv, b1, w2p):
    def body(s0_ref, s1_ref, u2_ref, dinv_ref, b1_ref, w2_ref, o_ref):
        dinv = dinv_ref[...]
        t = dinv * (s0_ref[...] + s1_ref[...] + u2_ref[...])
        h = jnp.maximum(t + b1_ref[...], 0.0)
        y2 = jnp.dot(h, w2_ref[...], preferred_element_type=jnp.float32)
        o_ref[...] = dinv * y2

    return _tc_call(
        body,
        [_row_spec(D_HID), _row_spec(D_HID, NB), _row_spec(D_HID), _row_spec(1),
         _full_spec((1, D_HID)), _full_spec((D_HID, 16))],
        jax.ShapeDtypeStruct((N_PAD, 16), jnp.float32),
        _row_spec(16),
    )(s, s, u2, dinv, b1, w2p)


def _tc_e(s, u4, dinv, b2p):
    def body(s0_ref, s1_ref, u4_ref, dinv_ref, b2_ref, o_ref):
        dinv = dinv_ref[...]
        o = dinv * (s0_ref[...] + s1_ref[...] + u4_ref[...]) + b2_ref[...]
        cols = lax.broadcasted_iota(jnp.int32, (BN, 16), 1)
        valid = cols < D_OUT
        m = jnp.max(jnp.where(valid, o, -jnp.inf), axis=1, keepdims=True)
        e = jnp.where(valid, jnp.exp(o - m), 0.0)
        lse = m + jnp.log(jnp.sum(e, axis=1, keepdims=True))
        o_ref[...] = o - lse

    return _tc_call(
        body,
        [_row_spec(16), _row_spec(16, NB), _row_spec(16), _row_spec(1),
         _full_spec((1, 16))],
        jax.ShapeDtypeStruct((N_PAD, 16), jnp.float32),
        _row_spec(16),
    )(s, s, u4, dinv, b2p)


def kernel(x, edge_index, W1, b1, W2, b2):
    src3 = edge_index[0].reshape(NW, NCH, B)
    dst3 = edge_index[1].reshape(NW, NCH, B)
    xp = jnp.zeros((N_PAD, D_IN), jnp.float32).at[:N].set(x)
    zeros64 = jnp.zeros((N_PAD, D_HID), jnp.float32)
    zeros16 = jnp.zeros((N_PAD, 16), jnp.float32)
    ones16 = jnp.ones((N_PAD, 16), jnp.float32)
    w2p = jnp.zeros((D_HID, 16), jnp.float32).at[:, :D_OUT].set(W2)
    b2p = jnp.zeros((1, 16), jnp.float32).at[0, :D_OUT].set(b2)

    spass64 = _make_spass(D_HID)
    spass16 = _make_spass(16)

    degp = spass16(src3, dst3, ones16, zeros16)
    dinv, u = _tc_a(degp, xp, W1)
    s1 = spass64(src3, dst3, u, zeros64)
    u2 = _tc_mid(s1, u, dinv, D_HID)
    s2 = spass64(src3, dst3, u2, zeros64)
    u3 = _tc_c(s2, u2, dinv, b1.reshape(1, D_HID), w2p)
    s3 = spass16(src3, dst3, u3, zeros16)
    u4 = _tc_mid(s3, u3, dinv, 16)
    s4 = spass16(src3, dst3, u4, zeros16)
    out = _tc_e(s4, u4, dinv, b2p)
    return out[:N, :D_OUT]


# speedup vs baseline: 38.7779x; 1.6345x over previous
import functools

import jax
import jax.numpy as jnp
from jax import lax
from jax.experimental import pallas as pl
from jax.experimental.pallas import tpu as pltpu
from jax.experimental.pallas import tpu_sc as plsc

N = 10000
E = 320000
D_IN, D_HID, D_OUT = 128, 64, 6
NC, NS = 2, 16
NW = NC * NS
N_PAD = 10240
ROWS_T = N_PAD // NS
E_W = E // NW
B = 250
NCH = E_W // B
U = 4
G = 2
BN = 1024
NB = N_PAD // BN


@functools.lru_cache(maxsize=None)
def _make_spass(d):
    mesh = plsc.VectorSubcoreMesh(
        core_axis_name="c", subcore_axis_name="s", num_cores=NC, num_subcores=NS
    )

    @functools.partial(
        pl.kernel,
        out_type=jax.ShapeDtypeStruct((NC * N_PAD, d), jnp.float32),
        mesh=mesh,
        compiler_params=pltpu.CompilerParams(use_tc_tiling_on_sc=False),
        scratch_types=[
            pltpu.VMEM((NCH, B), jnp.int32),
            pltpu.VMEM((NCH, B), jnp.int32),
            *([pltpu.VMEM((B, d), jnp.float32)] * U),
            *([pltpu.SemaphoreType.DMA] * U),
            *([pltpu.SemaphoreType.DMA] * U),
            pltpu.VMEM_SHARED((N_PAD, d), jnp.float32),
        ],
    )
    def spass(src3, dst3, rows, zeros, out, src_v, dst_v, *rest):
        bufs = rest[:U]
        gsem = rest[U:2 * U]
        ssem = rest[2 * U:3 * U]
        acc = rest[3 * U]
        c = lax.axis_index("c")
        s = lax.axis_index("s")
        wid = c * NS + s
        row0 = s * ROWS_T

        pltpu.sync_copy(zeros.at[pl.ds(row0, ROWS_T)], acc.at[pl.ds(row0, ROWS_T)])
        pltpu.sync_copy(src3.at[wid], src_v)
        pltpu.sync_copy(dst3.at[wid], dst_v)
        plsc.subcore_barrier()

        def gdesc(k, j):
            return pltpu.make_async_copy(rows.at[src_v.at[k]], bufs[j], gsem[j])

        def sdesc(k, j):
            return pltpu.make_async_copy(bufs[j], acc.at[dst_v.at[k]], ssem[j])

        for j in range(G):
            gdesc(j, j).start()

        def step(i, carry):
            for j in range(U):
                k = U * i + j
                gdesc(k, j).wait()
                sdesc(k, j).start(add=True)
                jj = (j + G) % U

                @pl.when(k >= U - G)
                def _():
                    sdesc(k - (U - G), jj).wait()

                @pl.when(k + G < NCH)
                def _():
                    gdesc(k + G, jj).start()

            return carry

        lax.fori_loop(0, NCH // U, step, 0)
        for t in range(U - G):
            k = NCH - (U - G) + t
            sdesc(k, k % U).wait()
        plsc.subcore_barrier()
        pltpu.sync_copy(
            acc.at[pl.ds(row0, ROWS_T)],
            out.at[pl.ds(c * N_PAD + row0, ROWS_T)],
        )

    return spass


def _row_spec(d, shift=0):
    return pl.BlockSpec((BN, d), lambda i, _s=shift: (i + _s, 0))


def _full_spec(shape):
    nd = len(shape)
    return pl.BlockSpec(shape, lambda i, _nd=nd: (0,) * nd)


def _tc_call(body, in_specs, out_shapes, out_specs):
    return pl.pallas_call(
        body,
        grid=(NB,),
        in_specs=in_specs,
        out_shape=out_shapes,
        out_specs=out_specs,
    )


def _tc_a(degp, x, w1):
    def body(p0_ref, p1_ref, x_ref, w1_ref, dinv_ref, u_ref):
        deg = 1.0 + p0_ref[:, 0:1] + p1_ref[:, 0:1]
        dinv = lax.rsqrt(deg)
        y1 = jnp.dot(x_ref[...], w1_ref[...], preferred_element_type=jnp.float32)
        dinv_ref[...] = dinv
        u_ref[...] = dinv * y1

    return _tc_call(
        body,
        [_row_spec(16), _row_spec(16, NB), _row_spec(D_IN), _full_spec((D_IN, D_HID))],
        (jax.ShapeDtypeStruct((N_PAD, 1), jnp.float32),
         jax.ShapeDtypeStruct((N_PAD, D_HID), jnp.float32)),
        (_row_spec(1), _row_spec(D_HID)),
    )(degp, degp, x, w1)


def _tc_mid(s, u, dinv, d):
    def body(s0_ref, s1_ref, u_ref, dinv_ref, o_ref):
        dinv = dinv_ref[...]
        o_ref[...] = dinv * dinv * (s0_ref[...] + s1_ref[...] + u_ref[...])

    return _tc_call(
        body,
        [_row_spec(d), _row_spec(d, NB), _row_spec(d), _row_spec(1)],
        jax.ShapeDtypeStruct((N_PAD, d), jnp.float32),
        _row_spec(d),
    )(s, s, u, dinv)


def _tc_c(s, u2, dinv, b1, w2p):
    def body(s0_ref, s1_ref, u2_ref, dinv_ref, b1_ref, w2_ref, o_ref):
        dinv = dinv_ref[...]
        t = dinv * (s0_ref[...] + s1_ref[...] + u2_ref[...])
        h = jnp.maximum(t + b1_ref[...], 0.0)
        y2 = jnp.dot(h, w2_ref[...], preferred_element_type=jnp.float32)
        o_ref[...] = dinv * y2

    return _tc_call(
        body,
        [_row_spec(D_HID), _row_spec(D_HID, NB), _row_spec(D_HID), _row_spec(1),
         _full_spec((1, D_HID)), _full_spec((D_HID, 16))],
        jax.ShapeDtypeStruct((N_PAD, 16), jnp.float32),
        _row_spec(16),
    )(s, s, u2, dinv, b1, w2p)


def _tc_e(s, u4, dinv, b2p):
    def body(s0_ref, s1_ref, u4_ref, dinv_ref, b2_ref, o_ref):
        dinv = dinv_ref[...]
        o = dinv * (s0_ref[...] + s1_ref[...] + u4_ref[...]) + b2_ref[...]
        cols = lax.broadcasted_iota(jnp.int32, (BN, 16), 1)
        valid = cols < D_OUT
        m = jnp.max(jnp.where(valid, o, -jnp.inf), axis=1, keepdims=True)
        e = jnp.where(valid, jnp.exp(o - m), 0.0)
        lse = m + jnp.log(jnp.sum(e, axis=1, keepdims=True))
        o_ref[...] = o - lse

    return _tc_call(
        body,
        [_row_spec(16), _row_spec(16, NB), _row_spec(16), _row_spec(1),
         _full_spec((1, 16))],
        jax.ShapeDtypeStruct((N_PAD, 16), jnp.float32),
        _row_spec(16),
    )(s, s, u4, dinv, b2p)


def kernel(x, edge_index, W1, b1, W2, b2):
    src3 = edge_index[0].reshape(NW, NCH, B)
    dst3 = edge_index[1].reshape(NW, NCH, B)
    xp = jnp.zeros((N_PAD, D_IN), jnp.float32).at[:N].set(x)
    zeros64 = jnp.zeros((N_PAD, D_HID), jnp.float32)
    zeros16 = jnp.zeros((N_PAD, 16), jnp.float32)
    ones16 = jnp.ones((N_PAD, 16), jnp.float32)
    w2p = jnp.zeros((D_HID, 16), jnp.float32).at[:, :D_OUT].set(W2)
    b2p = jnp.zeros((1, 16), jnp.float32).at[0, :D_OUT].set(b2)

    spass64 = _make_spass(D_HID)
    spass16 = _make_spass(16)

    degp = spass16(src3, dst3, ones16, zeros16)
    dinv, u = _tc_a(degp, xp, W1)
    s1 = spass64(src3, dst3, u, zeros64)
    u2 = _tc_mid(s1, u, dinv, D_HID)
    s2 = spass64(src3, dst3, u2, zeros64)
    u3 = _tc_c(s2, u2, dinv, b1.reshape(1, D_HID), w2p)
    s3 = spass16(src3, dst3, u3, zeros16)
    u4 = _tc_mid(s3, u3, dinv, 16)
    s4 = spass16(src3, dst3, u4, zeros16)
    out = _tc_e(s4, u4, dinv, b2p)
    return out[:N, :D_OUT]


# speedup vs baseline: 41.8602x; 1.0795x over previous
import functools

import jax
import jax.numpy as jnp
from jax import lax
from jax.experimental import pallas as pl
from jax.experimental.pallas import tpu as pltpu
from jax.experimental.pallas import tpu_sc as plsc

N = 10000
E = 320000
D_IN, D_HID, D_OUT = 128, 64, 6
NC, NS = 2, 16
NW = NC * NS
N_PAD = 10240
ROWS_T = N_PAD // NS
E_W = E // NW
_CHUNK_CFG = {64: (200, 5, 3), 16: (500, 5, 3)}
BN = 1024
NB = N_PAD // BN


@functools.lru_cache(maxsize=None)
def _make_spass(d):
    B, U, G = _CHUNK_CFG[d]
    NCH = E_W // B
    assert NCH % U == 0 and E_W % B == 0
    mesh = plsc.VectorSubcoreMesh(
        core_axis_name="c", subcore_axis_name="s", num_cores=NC, num_subcores=NS
    )

    @functools.partial(
        pl.kernel,
        out_type=jax.ShapeDtypeStruct((NC * N_PAD, d), jnp.float32),
        mesh=mesh,
        compiler_params=pltpu.CompilerParams(use_tc_tiling_on_sc=False),
        scratch_types=[
            pltpu.VMEM((NCH, B), jnp.int32),
            pltpu.VMEM((NCH, B), jnp.int32),
            *([pltpu.VMEM((B, d), jnp.float32)] * U),
            *([pltpu.SemaphoreType.DMA] * U),
            *([pltpu.SemaphoreType.DMA] * U),
            pltpu.VMEM_SHARED((N_PAD, d), jnp.float32),
        ],
    )
    def spass(src3, dst3, rows, zeros, out, src_v, dst_v, *rest):
        bufs = rest[:U]
        gsem = rest[U:2 * U]
        ssem = rest[2 * U:3 * U]
        acc = rest[3 * U]
        c = lax.axis_index("c")
        s = lax.axis_index("s")
        wid = c * NS + s
        row0 = s * ROWS_T

        pltpu.sync_copy(zeros.at[pl.ds(row0, ROWS_T)], acc.at[pl.ds(row0, ROWS_T)])
        pltpu.sync_copy(src3.at[wid], src_v)
        pltpu.sync_copy(dst3.at[wid], dst_v)
        plsc.subcore_barrier()

        def gdesc(k, j):
            return pltpu.make_async_copy(rows.at[src_v.at[k]], bufs[j], gsem[j])

        def sdesc(k, j):
            return pltpu.make_async_copy(bufs[j], acc.at[dst_v.at[k]], ssem[j])

        for j in range(G):
            gdesc(j, j).start()

        def step(i, carry):
            for j in range(U):
                k = U * i + j
                gdesc(k, j).wait()
                sdesc(k, j).start(add=True)
                jj = (j + G) % U

                @pl.when(k >= U - G)
                def _():
                    sdesc(k - (U - G), jj).wait()

                @pl.when(k + G < NCH)
                def _():
                    gdesc(k + G, jj).start()

            return carry

        lax.fori_loop(0, NCH // U, step, 0)
        for t in range(U - G):
            k = NCH - (U - G) + t
            sdesc(k, k % U).wait()
        plsc.subcore_barrier()
        pltpu.sync_copy(
            acc.at[pl.ds(row0, ROWS_T)],
            out.at[pl.ds(c * N_PAD + row0, ROWS_T)],
        )

    return spass


def _row_spec(d, shift=0):
    return pl.BlockSpec((BN, d), lambda i, _s=shift: (i + _s, 0))


def _full_spec(shape):
    nd = len(shape)
    return pl.BlockSpec(shape, lambda i, _nd=nd: (0,) * nd)


def _tc_call(body, in_specs, out_shapes, out_specs):
    return pl.pallas_call(
        body,
        grid=(NB,),
        in_specs=in_specs,
        out_shape=out_shapes,
        out_specs=out_specs,
    )


def _tc_a(degp, x, w1):
    def body(p0_ref, p1_ref, x_ref, w1_ref, dinv_ref, u_ref):
        deg = 1.0 + p0_ref[:, 0:1] + p1_ref[:, 0:1]
        dinv = lax.rsqrt(deg)
        y1 = jnp.dot(x_ref[...], w1_ref[...], preferred_element_type=jnp.float32)
        dinv_ref[...] = dinv
        u_ref[...] = dinv * y1

    return _tc_call(
        body,
        [_row_spec(16), _row_spec(16, NB), _row_spec(D_IN), _full_spec((D_IN, D_HID))],
        (jax.ShapeDtypeStruct((N_PAD, 1), jnp.float32),
         jax.ShapeDtypeStruct((N_PAD, D_HID), jnp.float32)),
        (_row_spec(1), _row_spec(D_HID)),
    )(degp, degp, x, w1)


def _tc_mid(s, u, dinv, d):
    def body(s0_ref, s1_ref, u_ref, dinv_ref, o_ref):
        dinv = dinv_ref[...]
        o_ref[...] = dinv * dinv * (s0_ref[...] + s1_ref[...] + u_ref[...])

    return _tc_call(
        body,
        [_row_spec(d), _row_spec(d, NB), _row_spec(d), _row_spec(1)],
        jax.ShapeDtypeStruct((N_PAD, d), jnp.float32),
        _row_spec(d),
    )(s, s, u, dinv)


def _tc_c(s, u2, dinv, b1, w2p):
    def body(s0_ref, s1_ref, u2_ref, dinv_ref, b1_ref, w2_ref, o_ref):
        dinv = dinv_ref[...]
        t = dinv * (s0_ref[...] + s1_ref[...] + u2_ref[...])
        h = jnp.maximum(t + b1_ref[...], 0.0)
        y2 = jnp.dot(h, w2_ref[...], preferred_element_type=jnp.float32)
        o_ref[...] = dinv * y2

    return _tc_call(
        body,
        [_row_spec(D_HID), _row_spec(D_HID, NB), _row_spec(D_HID), _row_spec(1),
         _full_spec((1, D_HID)), _full_spec((D_HID, 16))],
        jax.ShapeDtypeStruct((N_PAD, 16), jnp.float32),
        _row_spec(16),
    )(s, s, u2, dinv, b1, w2p)


def _tc_e(s, u4, dinv, b2p):
    def body(s0_ref, s1_ref, u4_ref, dinv_ref, b2_ref, o_ref):
        dinv = dinv_ref[...]
        o = dinv * (s0_ref[...] + s1_ref[...] + u4_ref[...]) + b2_ref[...]
        cols = lax.broadcasted_iota(jnp.int32, (BN, 16), 1)
        valid = cols < D_OUT
        m = jnp.max(jnp.where(valid, o, -jnp.inf), axis=1, keepdims=True)
        e = jnp.where(valid, jnp.exp(o - m), 0.0)
        lse = m + jnp.log(jnp.sum(e, axis=1, keepdims=True))
        o_ref[...] = o - lse

    return _tc_call(
        body,
        [_row_spec(16), _row_spec(16, NB), _row_spec(16), _row_spec(1),
         _full_spec((1, 16))],
        jax.ShapeDtypeStruct((N_PAD, 16), jnp.float32),
        _row_spec(16),
    )(s, s, u4, dinv, b2p)


def kernel(x, edge_index, W1, b1, W2, b2):
    def idx3(d):
        B, _, _ = _CHUNK_CFG[d]
        shape = (NW, E_W // B, B)
        return edge_index[0].reshape(shape), edge_index[1].reshape(shape)

    src64, dst64 = idx3(64)
    src16, dst16 = idx3(16)
    xp = jnp.zeros((N_PAD, D_IN), jnp.float32).at[:N].set(x)
    zeros64 = jnp.zeros((N_PAD, D_HID), jnp.float32)
    zeros16 = jnp.zeros((N_PAD, 16), jnp.float32)
    ones16 = jnp.ones((N_PAD, 16), jnp.float32)
    w2p = jnp.zeros((D_HID, 16), jnp.float32).at[:, :D_OUT].set(W2)
    b2p = jnp.zeros((1, 16), jnp.float32).at[0, :D_OUT].set(b2)

    spass64 = _make_spass(D_HID)
    spass16 = _make_spass(16)

    degp = spass16(src16, dst16, ones16, zeros16)
    dinv, u = _tc_a(degp, xp, W1)
    s1 = spass64(src64, dst64, u, zeros64)
    u2 = _tc_mid(s1, u, dinv, D_HID)
    s2 = spass64(src64, dst64, u2, zeros64)
    u3 = _tc_c(s2, u2, dinv, b1.reshape(1, D_HID), w2p)
    s3 = spass16(src16, dst16, u3, zeros16)
    u4 = _tc_mid(s3, u3, dinv, 16)
    s4 = spass16(src16, dst16, u4, zeros16)
    out = _tc_e(s4, u4, dinv, b2p)
    return out[:N, :D_OUT]
